# Initial kernel scaffold; baseline (speedup 1.0000x reference)
import jax
import jax.numpy as jnp
from jax.experimental import pallas as pl


def kernel(xyz, points, wn_w0, wn_b0, wn_w1, wn_b1, wn_w2, wn_b2, lin_w, lin_b):
    raise NotImplementedError("write your pallas kernel here")


# speedup vs baseline: 1.0002x; 1.0002x over previous
import jax
import jax.numpy as jnp
import numpy as np
from jax.experimental import pallas as pl

B, N = 8, 4096
NPOINT = 1024
NSAMPLE = 32
C_PTS = 64
IN_CHANNEL = 67
OUT_CHANNEL = 64
WEIGHTNET = 16


def _square_distance(src, dst):
    d = -2.0 * jnp.matmul(src, jnp.transpose(dst, (0, 2, 1)))
    d = d + jnp.sum(src ** 2, axis=-1)[:, :, None]
    d = d + jnp.sum(dst ** 2, axis=-1)[:, None, :]
    return d


def _fps(xyz, npoint):
    b, n, _ = xyz.shape

    def body(i, state):
        dists, farthest, idxs = state
        idxs = idxs.at[:, i].set(farthest)
        centroid = xyz[jnp.arange(b), farthest]
        d = jnp.sum((xyz - centroid[:, None, :]) ** 2, axis=-1)
        dists = jnp.minimum(dists, d)
        farthest = jnp.argmax(dists, axis=-1).astype(jnp.int32)
        return dists, farthest, idxs

    init = (jnp.full((b, n), 1e10, dtype=xyz.dtype),
            jnp.zeros((b,), dtype=jnp.int32),
            jnp.zeros((b, npoint), dtype=jnp.int32))
    _, _, idxs = jax.lax.fori_loop(0, npoint, body, init)
    return idxs


def _gather(points, idx):
    return jax.vmap(lambda p, i: p[i])(points, idx)


def _leaky_kernel(x_ref, o_ref):
    x = x_ref[...]
    o_ref[...] = jnp.where(x > 0, x, 0.1 * x)


def kernel(xyz, points, wn_w0, wn_b0, wn_w1, wn_b1, wn_w2, wn_b2, lin_w, lin_b):
    b = xyz.shape[0]
    xyz_t = jnp.transpose(xyz, (0, 2, 1))
    pts_t = jnp.transpose(points, (0, 2, 1))
    fps_idx = _fps(jax.lax.stop_gradient(xyz_t), NPOINT)
    new_xyz = _gather(xyz_t, fps_idx)
    sqrdists = _square_distance(new_xyz, xyz_t)
    _, knn_idx = jax.lax.top_k(-jax.lax.stop_gradient(sqrdists), NSAMPLE)
    grouped_xyz = _gather(xyz_t, knn_idx.reshape(b, -1)).reshape(b, NPOINT, NSAMPLE, 3)
    grouped_xyz_norm = grouped_xyz - new_xyz[:, :, None, :]
    grouped_points = _gather(pts_t, knn_idx.reshape(b, -1)).reshape(b, NPOINT, NSAMPLE, C_PTS)
    new_points = jnp.concatenate([grouped_xyz_norm, grouped_points], axis=-1)
    w = jax.nn.relu(jnp.einsum('bsnc,oc->bsno', grouped_xyz_norm, wn_w0) + wn_b0)
    w = jax.nn.relu(jnp.einsum('bsnc,oc->bsno', w, wn_w1) + wn_b1)
    w = jax.nn.relu(jnp.einsum('bsnc,oc->bsno', w, wn_w2) + wn_b2)
    feat = jnp.matmul(jnp.transpose(new_points, (0, 1, 3, 2)), w)
    feat = feat.reshape(b, NPOINT, -1)
    out = jnp.matmul(feat, lin_w.T) + lin_b
    out = jnp.transpose(out, (0, 2, 1))
    out = pl.pallas_call(
        _leaky_kernel,
        out_shape=jax.ShapeDtypeStruct(out.shape, out.dtype),
    )(out)
    return jnp.transpose(new_xyz, (0, 2, 1)), out, fps_idx


# speedup vs baseline: 1.6230x; 1.6226x over previous
import jax
import jax.numpy as jnp
import numpy as np
from jax.experimental import pallas as pl

B, N = 8, 4096
NPOINT = 1024
NSAMPLE = 32
C_PTS = 64
IN_CHANNEL = 67
OUT_CHANNEL = 64
WEIGHTNET = 16


def _square_distance(src, dst):
    d = -2.0 * jnp.matmul(src, jnp.transpose(dst, (0, 2, 1)))
    d = d + jnp.sum(src ** 2, axis=-1)[:, :, None]
    d = d + jnp.sum(dst ** 2, axis=-1)[:, None, :]
    return d


def _fps(xyz, npoint):
    b, n, _ = xyz.shape

    def body(i, state):
        dists, farthest, idxs = state
        idxs = idxs.at[:, i].set(farthest)
        centroid = xyz[jnp.arange(b), farthest]
        d = jnp.sum((xyz - centroid[:, None, :]) ** 2, axis=-1)
        dists = jnp.minimum(dists, d)
        farthest = jnp.argmax(dists, axis=-1).astype(jnp.int32)
        return dists, farthest, idxs

    init = (jnp.full((b, n), 1e10, dtype=xyz.dtype),
            jnp.zeros((b,), dtype=jnp.int32),
            jnp.zeros((b, npoint), dtype=jnp.int32))
    _, _, idxs = jax.lax.fori_loop(0, npoint, body, init)
    return idxs


def _gather(points, idx):
    return jax.vmap(lambda p, i: p[i])(points, idx)


def _leaky_kernel(x_ref, o_ref):
    x = x_ref[...]
    o_ref[...] = jnp.where(x > 0, x, 0.1 * x)


def _fps_kernel(x_ref, idx_ref, cen_ref):
    x = x_ref[0]
    y = x_ref[1]
    z = x_ref[2]
    lane = jax.lax.broadcasted_iota(jnp.int32, x.shape, 1)
    out_lane = jax.lax.broadcasted_iota(jnp.int32, (x.shape[0], NPOINT), 1)

    zero_out = out_lane * 0
    idx_ref[...] = zero_out
    cen_ref[0] = zero_out.astype(jnp.float32)
    cen_ref[1] = zero_out.astype(jnp.float32)
    cen_ref[2] = zero_out.astype(jnp.float32)

    def body(i, state):
        dists, far = state
        sel = (out_lane == i).astype(jnp.float32)
        idx_ref[...] = idx_ref[...] + (sel * far.astype(jnp.float32)).astype(jnp.int32)
        onehot = lane == far
        zero = jnp.zeros_like(x)
        cx = jnp.sum(jnp.where(onehot, x, zero), axis=1, keepdims=True)
        cy = jnp.sum(jnp.where(onehot, y, zero), axis=1, keepdims=True)
        cz = jnp.sum(jnp.where(onehot, z, zero), axis=1, keepdims=True)
        cen_ref[0] = cen_ref[0] + sel * cx
        cen_ref[1] = cen_ref[1] + sel * cy
        cen_ref[2] = cen_ref[2] + sel * cz
        dx = x - cx
        dy = y - cy
        dz = z - cz
        d = dx * dx + dy * dy + dz * dz
        dists = jnp.minimum(dists, d)
        m = jnp.max(dists, axis=1, keepdims=True)
        cand = jnp.where(dists == m, lane, jnp.int32(N))
        far = jnp.min(cand, axis=1, keepdims=True)
        return dists, far

    init = (x * 0.0 + 1e10,
            jnp.min(lane * 0, axis=1, keepdims=True))
    jax.lax.fori_loop(0, NPOINT, body, init)


def _run_fps(xyz):
    x3 = jnp.transpose(xyz, (1, 0, 2))
    idx, cen = pl.pallas_call(
        _fps_kernel,
        out_shape=(
            jax.ShapeDtypeStruct((B, NPOINT), jnp.int32),
            jax.ShapeDtypeStruct((3, B, NPOINT), jnp.float32),
        ),
    )(x3)
    return idx, jnp.transpose(cen, (1, 2, 0))


def kernel(xyz, points, wn_w0, wn_b0, wn_w1, wn_b1, wn_w2, wn_b2, lin_w, lin_b):
    b = xyz.shape[0]
    xyz_t = jnp.transpose(xyz, (0, 2, 1))
    pts_t = jnp.transpose(points, (0, 2, 1))
    fps_idx, _ = _run_fps(xyz)
    new_xyz = _gather(xyz_t, fps_idx)
    sqrdists = _square_distance(new_xyz, xyz_t)
    _, knn_idx = jax.lax.top_k(-jax.lax.stop_gradient(sqrdists), NSAMPLE)
    grouped_xyz = _gather(xyz_t, knn_idx.reshape(b, -1)).reshape(b, NPOINT, NSAMPLE, 3)
    grouped_xyz_norm = grouped_xyz - new_xyz[:, :, None, :]
    grouped_points = _gather(pts_t, knn_idx.reshape(b, -1)).reshape(b, NPOINT, NSAMPLE, C_PTS)
    new_points = jnp.concatenate([grouped_xyz_norm, grouped_points], axis=-1)
    w = jax.nn.relu(jnp.einsum('bsnc,oc->bsno', grouped_xyz_norm, wn_w0) + wn_b0)
    w = jax.nn.relu(jnp.einsum('bsnc,oc->bsno', w, wn_w1) + wn_b1)
    w = jax.nn.relu(jnp.einsum('bsnc,oc->bsno', w, wn_w2) + wn_b2)
    feat = jnp.matmul(jnp.transpose(new_points, (0, 1, 3, 2)), w)
    feat = feat.reshape(b, NPOINT, -1)
    out = jnp.matmul(feat, lin_w.T) + lin_b
    out = jnp.transpose(out, (0, 2, 1))
    out = pl.pallas_call(
        _leaky_kernel,
        out_shape=jax.ShapeDtypeStruct(out.shape, out.dtype),
    )(out)
    return jnp.transpose(new_xyz, (0, 2, 1)), out, fps_idx


# speedup vs baseline: 6.9179x; 4.2624x over previous
import jax
import jax.numpy as jnp
import numpy as np
from jax.experimental import pallas as pl
from jax.experimental.pallas import tpu as pltpu

B, N = 8, 4096
NPOINT = 1024
NSAMPLE = 32
C_PTS = 64
IN_CHANNEL = 67
OUT_CHANNEL = 64
WEIGHTNET = 16


def _fps_kernel(x_ref, idx_ref, cen_ref):
    x = x_ref[0]
    y = x_ref[1]
    z = x_ref[2]
    lane = jax.lax.broadcasted_iota(jnp.int32, x.shape, 1)
    out_lane = jax.lax.broadcasted_iota(jnp.int32, (x.shape[0], NPOINT), 1)

    zero_out = out_lane * 0
    idx_ref[...] = zero_out
    cen_ref[0] = zero_out.astype(jnp.float32)
    cen_ref[1] = zero_out.astype(jnp.float32)
    cen_ref[2] = zero_out.astype(jnp.float32)

    def body(i, state):
        dists, far = state
        sel = (out_lane == i).astype(jnp.float32)
        idx_ref[...] = idx_ref[...] + (sel * far.astype(jnp.float32)).astype(jnp.int32)
        onehot = lane == far
        zero = jnp.zeros_like(x)
        cx = jnp.sum(jnp.where(onehot, x, zero), axis=1, keepdims=True)
        cy = jnp.sum(jnp.where(onehot, y, zero), axis=1, keepdims=True)
        cz = jnp.sum(jnp.where(onehot, z, zero), axis=1, keepdims=True)
        cen_ref[0] = cen_ref[0] + sel * cx
        cen_ref[1] = cen_ref[1] + sel * cy
        cen_ref[2] = cen_ref[2] + sel * cz
        dx = x - cx
        dy = y - cy
        dz = z - cz
        d = dx * dx + dy * dy + dz * dz
        dists = jnp.minimum(dists, d)
        m = jnp.max(dists, axis=1, keepdims=True)
        cand = jnp.where(dists == m, lane, jnp.int32(N))
        far = jnp.min(cand, axis=1, keepdims=True)
        return dists, far

    init = (x * 0.0 + 1e10,
            jnp.min(lane * 0, axis=1, keepdims=True))
    jax.lax.fori_loop(0, NPOINT, body, init)


def _run_fps(xyz):
    x3 = jnp.transpose(xyz, (1, 0, 2))
    idx, cen = pl.pallas_call(
        _fps_kernel,
        out_shape=(
            jax.ShapeDtypeStruct((B, NPOINT), jnp.int32),
            jax.ShapeDtypeStruct((3, B, NPOINT), jnp.float32),
        ),
    )(x3)
    return idx, jnp.transpose(cen, (1, 2, 0))


S_TILE = 128
N_TILES = NPOINT // S_TILE
FPAD = 128


def _conv_kernel(cen_ref, x3_ref, all_ref, w0_ref, b0_ref, w1_ref, b1_ref,
                 w2_ref, b2_ref, lw_ref, lb_ref, out_ref, d_ref, feat_ref):
    c = cen_ref[0]
    x3 = x3_ref[0]
    allf = all_ref[0]
    lane = jax.lax.broadcasted_iota(jnp.int32, (S_TILE, N), 1)

    dot = jax.lax.dot_general(c, x3, (((1,), (0,)), ((), ())))
    csq = jnp.sum(c * c, axis=1, keepdims=True)
    xsq = jnp.sum(x3 * x3, axis=0, keepdims=True)
    d_ref[...] = (-2.0 * dot + csq) + xsq

    feat_ref[...] = jax.lax.broadcasted_iota(
        jnp.int32, (S_TILE, WEIGHTNET * FPAD), 1).astype(jnp.float32) * 0.0

    r3 = jax.lax.broadcasted_iota(jnp.int32, (3, FPAD), 0)
    c3 = jax.lax.broadcasted_iota(jnp.int32, (3, FPAD), 1)
    eye3 = (r3 == c3).astype(jnp.float32)
    cpad = jax.lax.dot_general(c, eye3, (((1,), (0,)), ((), ())))

    w0 = w0_ref[...]
    w1 = w1_ref[...]
    w2 = w2_ref[...]
    b0 = b0_ref[...]
    b1 = b1_ref[...]
    b2 = b2_ref[...]

    def body(k, carry):
        dcur = d_ref[...]
        m = jnp.min(dcur, axis=1, keepdims=True)
        eq = dcur == m
        fi = jnp.min(jnp.where(eq, lane, jnp.int32(N)), axis=1, keepdims=True)
        onehot = lane == fi
        d_ref[...] = jnp.where(onehot, jnp.float32(3e38), dcur)
        oh_f = onehot.astype(jnp.float32)
        g = jax.lax.dot_general(oh_f, allf, (((1,), (0,)), ((), ())))
        gn = g - cpad
        n3 = gn[:, 0:3]
        h = jnp.maximum(jax.lax.dot_general(n3, w0, (((1,), (0,)), ((), ()))) + b0, 0.0)
        h = jnp.maximum(jax.lax.dot_general(h, w1, (((1,), (0,)), ((), ()))) + b1, 0.0)
        wk = jnp.maximum(jax.lax.dot_general(h, w2, (((1,), (0,)), ((), ()))) + b2, 0.0)
        acc = feat_ref[...]
        upd = []
        for o in range(WEIGHTNET):
            upd.append(gn * wk[:, o:o + 1])
        feat_ref[...] = acc + jnp.concatenate(upd, axis=1)
        return carry

    jax.lax.fori_loop(0, NSAMPLE, body, 0)

    feat = feat_ref[...]
    res = jax.lax.dot_general(feat, lw_ref[...], (((1,), (0,)), ((), ()))) + lb_ref[...]
    out_ref[0] = jnp.where(res > 0, res, 0.1 * res)


def _run_conv(cen_t, xyz, allf, wn_w0, wn_b0, wn_w1, wn_b1, wn_w2, wn_b2,
              lw_pad, lin_b):
    grid = (B, N_TILES)
    return pl.pallas_call(
        _conv_kernel,
        grid=grid,
        in_specs=[
            pl.BlockSpec((1, S_TILE, 3), lambda b, t: (b, t, 0)),
            pl.BlockSpec((1, 3, N), lambda b, t: (b, 0, 0)),
            pl.BlockSpec((1, N, FPAD), lambda b, t: (b, 0, 0)),
            pl.BlockSpec((3, 8), lambda b, t: (0, 0)),
            pl.BlockSpec((1, 8), lambda b, t: (0, 0)),
            pl.BlockSpec((8, 8), lambda b, t: (0, 0)),
            pl.BlockSpec((1, 8), lambda b, t: (0, 0)),
            pl.BlockSpec((8, WEIGHTNET), lambda b, t: (0, 0)),
            pl.BlockSpec((1, WEIGHTNET), lambda b, t: (0, 0)),
            pl.BlockSpec((WEIGHTNET * FPAD, OUT_CHANNEL), lambda b, t: (0, 0)),
            pl.BlockSpec((1, OUT_CHANNEL), lambda b, t: (0, 0)),
        ],
        out_specs=pl.BlockSpec((1, S_TILE, OUT_CHANNEL), lambda b, t: (b, t, 0)),
        out_shape=jax.ShapeDtypeStruct((B, NPOINT, OUT_CHANNEL), jnp.float32),
        scratch_shapes=[
            pltpu.VMEM((S_TILE, N), jnp.float32),
            pltpu.VMEM((S_TILE, WEIGHTNET * FPAD), jnp.float32),
        ],
    )(cen_t, xyz, allf, wn_w0, wn_b0, wn_w1, wn_b1, wn_w2, wn_b2, lw_pad, lin_b)


def kernel(xyz, points, wn_w0, wn_b0, wn_w1, wn_b1, wn_w2, wn_b2, lin_w, lin_b):
    b = xyz.shape[0]
    xyz_t = jnp.transpose(xyz, (0, 2, 1))
    pts_t = jnp.transpose(points, (0, 2, 1))
    fps_idx, cen = _run_fps(xyz)
    allf = jnp.concatenate(
        [xyz_t, pts_t, jnp.zeros((b, N, FPAD - IN_CHANNEL), jnp.float32)], axis=2)
    lwp = jnp.zeros((WEIGHTNET, FPAD, OUT_CHANNEL), jnp.float32)
    lw3 = lin_w.reshape(OUT_CHANNEL, IN_CHANNEL, WEIGHTNET)
    lwp = lwp.at[:, :IN_CHANNEL, :].set(jnp.transpose(lw3, (2, 1, 0)))
    lwp = lwp.reshape(WEIGHTNET * FPAD, OUT_CHANNEL)
    out = _run_conv(cen, xyz, allf, wn_w0.T, wn_b0[None], wn_w1.T, wn_b1[None],
                    wn_w2.T, wn_b2[None], lwp, lin_b[None])
    return jnp.transpose(cen, (0, 2, 1)), jnp.transpose(out, (0, 2, 1)), fps_idx
